# baseline (speedup 1.0000x reference)
import functools

import jax
import jax.numpy as jnp
from jax import lax
from jax.experimental import pallas as pl
from jax.experimental.pallas import tpu as pltpu
from jax.experimental.pallas import tpu_sc as plsc

_BG = (0, 1, 2, 3, 4, 8, 9, 10)
_NBG = 8
_F = 2048
_HW = 8192
_NPOS = 16384
_FB = 128
_NG = 2
_NH = 2
_GW = _F // _NG
_HH = _HW // _NH

_ROWS_PER_BATCH = 4096
_CHUNK = 512
_NW = 32
_L = 16


def _sc_labels_body(gs_hbm, gt_hbm, out_hbm, idx_v, rows_v, lab_v):
    wid = lax.axis_index("s") * 2 + lax.axis_index("c")
    p0 = wid * _CHUNK
    iota = lax.broadcasted_iota(jnp.int32, (_L,), 0)
    for d in range(2):
        src = gs_hbm if d == 0 else gt_hbm
        for n in range(2):
            q = p0 + 16 * (n * _L + iota)
            b = q >> 13
            i = (q >> 7) & 63
            jc = (q >> 4) & 7
            idx_v[pl.ds(n * _L, _L)] = (b << 12) + (i << 6) + jc
        pltpu.sync_copy(src.at[idx_v], rows_v)
        for m in range(_CHUNK // _L):
            row = jnp.full((_L,), m, jnp.int32)
            lab_v[pl.ds(m * _L, _L)] = plsc.load_gather(rows_v, [row, 8 * iota])
        pltpu.sync_copy(lab_v, out_hbm.at[pl.ds(d * _NPOS + p0, _CHUNK)])


def _sc_labels(gs_rows, gt_rows):
    mesh = plsc.VectorSubcoreMesh(core_axis_name="c", subcore_axis_name="s")
    kern = pl.kernel(
        _sc_labels_body,
        mesh=mesh,
        compiler_params=pltpu.CompilerParams(needs_layout_passes=False),
        out_type=jax.ShapeDtypeStruct((2 * _NPOS,), jnp.int32),
        scratch_types=[
            pltpu.VMEM((32,), jnp.int32),
            pltpu.VMEM((32, 128), jnp.int32),
            pltpu.VMEM((_CHUNK,), jnp.int32),
        ],
    )
    return kern(gs_rows, gt_rows)


def _tc_body(lab_ref, *refs):
    feat_refs = refs[:2 * _NG * _NH]
    out_refs = refs[2 * _NG * _NH:2 * _NG * _NH + _NG]
    cnt_ref = refs[-1]
    b = pl.program_id(1)
    fb = pl.program_id(0)
    cls = lax.broadcasted_iota(jnp.int32, (_NBG, 1), 0)
    bg = jnp.where(cls < 5, cls, cls + 3)
    dims = (((1,), (1,)), ((), ()))
    for d in range(2):
        labs = lab_ref[pl.ds(2 * d + b, 1), 0, :]
        onehot = (labs == bg).astype(jnp.float32)
        cntb = jnp.sum(onehot, axis=1, keepdims=True)

        @pl.when(b == 0)
        def _():
            cnt_ref[d] = cntb

        @pl.when(b == 1)
        def _():
            cnt_ref[d] = cnt_ref[d] + cntb

        col = pl.ds(fb * _FB, _FB)
        for g in range(_NG):
            contrib = jnp.zeros((_NBG, _FB), jnp.float32)
            for h in range(_NH):
                feat = feat_refs[(d * _NG + g) * _NH + h][0]
                feat = feat.reshape(_FB, _HH)
                oh = onehot[:, h * _HH:(h + 1) * _HH]
                contrib += lax.dot_general(
                    oh, feat, dims, preferred_element_type=jnp.float32)

            @pl.when(b == 0)
            def _():
                out_refs[g][d, :, col] = contrib

            @pl.when(b == 1)
            def _():
                acc = out_refs[g][d, :, col] + contrib
                cnt = cnt_ref[d]
                out_refs[g][d, :, col] = jnp.where(cnt > 0.0,
                                                   acc / jnp.maximum(cnt, 1.0),
                                                   jnp.inf)


def _tc_reduce(labels, fs, ft):
    nfb = _GW // _FB
    feat_specs = []
    for d in range(2):
        for g in range(_NG):
            for h in range(_NH):
                feat_specs.append(
                    pl.BlockSpec((1, _FB, _HH),
                                 lambda fb, b, g=g, h=h:
                                 (b, g * nfb + fb, h)))
    outs = pl.pallas_call(
        _tc_body,
        grid=(nfb, 2),
        in_specs=[pl.BlockSpec((4, 1, _HW), lambda fb, b: (0, 0, 0))]
        + feat_specs,
        out_specs=[pl.BlockSpec((2, _NBG, _GW), lambda fb, b: (0, 0, 0))
                   for _ in range(_NG)],
        out_shape=[jax.ShapeDtypeStruct((2, _NBG, _GW), jnp.float32)
                   for _ in range(_NG)],
        scratch_shapes=[pltpu.VMEM((2, _NBG, 1), jnp.float32)],
        compiler_params=pltpu.CompilerParams(
            dimension_semantics=("parallel", "arbitrary")),
    )(labels, *([fs] * _NG * _NH), *([ft] * _NG * _NH))
    return jnp.concatenate(outs, axis=2)


def kernel(feat_source, gt_source, feat_target, gt_target):
    fs = feat_source.reshape(2, _F, _HW)
    ft = feat_target.reshape(2, _F, _HW)
    gs_rows = gt_source.astype(jnp.int32).reshape(2 * _ROWS_PER_BATCH, 128)
    gt_rows = gt_target.astype(jnp.int32).reshape(2 * _ROWS_PER_BATCH, 128)
    labels = _sc_labels(gs_rows, gt_rows).reshape(4, 1, _HW)
    return _tc_reduce(labels, fs, ft)


# speedup vs baseline: 1.0095x; 1.0095x over previous
import functools

import jax
import jax.numpy as jnp
from jax import lax
from jax.experimental import pallas as pl
from jax.experimental.pallas import tpu as pltpu
from jax.experimental.pallas import tpu_sc as plsc

_BG = (0, 1, 2, 3, 4, 8, 9, 10)
_NBG = 8
_F = 2048
_HW = 8192
_NPOS = 16384
_FB = 128
_NG = 2
_NH = 4
_GW = _F // _NG
_HH = _HW // _NH

_ROWS_PER_BATCH = 4096
_CHUNK = 512
_NW = 32
_L = 16


def _sc_labels_body(gs_hbm, gt_hbm, out_hbm, idx_v, rows_v, lab_v):
    wid = lax.axis_index("s") * 2 + lax.axis_index("c")
    p0 = wid * _CHUNK
    iota = lax.broadcasted_iota(jnp.int32, (_L,), 0)
    for d in range(2):
        src = gs_hbm if d == 0 else gt_hbm
        for n in range(2):
            q = p0 + 16 * (n * _L + iota)
            b = q >> 13
            i = (q >> 7) & 63
            jc = (q >> 4) & 7
            idx_v[pl.ds(n * _L, _L)] = (b << 12) + (i << 6) + jc
        pltpu.sync_copy(src.at[idx_v], rows_v)
        for m in range(_CHUNK // _L):
            row = jnp.full((_L,), m, jnp.int32)
            lab_v[pl.ds(m * _L, _L)] = plsc.load_gather(rows_v, [row, 8 * iota])
        pltpu.sync_copy(lab_v, out_hbm.at[pl.ds(d * _NPOS + p0, _CHUNK)])


def _sc_labels(gs_rows, gt_rows):
    mesh = plsc.VectorSubcoreMesh(core_axis_name="c", subcore_axis_name="s")
    kern = pl.kernel(
        _sc_labels_body,
        mesh=mesh,
        compiler_params=pltpu.CompilerParams(needs_layout_passes=False),
        out_type=jax.ShapeDtypeStruct((2 * _NPOS,), jnp.int32),
        scratch_types=[
            pltpu.VMEM((32,), jnp.int32),
            pltpu.VMEM((32, 128), jnp.int32),
            pltpu.VMEM((_CHUNK,), jnp.int32),
        ],
    )
    return kern(gs_rows, gt_rows)


def _tc_body(lab_ref, *refs):
    feat_refs = refs[:2 * _NG * _NH]
    out_refs = refs[2 * _NG * _NH:2 * _NG * _NH + _NG]
    cnt_ref = refs[-1]
    b = pl.program_id(1)
    fb = pl.program_id(0)
    cls = lax.broadcasted_iota(jnp.int32, (_NBG, 1), 0)
    bg = jnp.where(cls < 5, cls, cls + 3)
    dims = (((1,), (1,)), ((), ()))
    for d in range(2):
        labs = lab_ref[pl.ds(2 * d + b, 1), 0, :]
        onehot = (labs == bg).astype(jnp.float32)
        cntb = jnp.sum(onehot, axis=1, keepdims=True)

        @pl.when(b == 0)
        def _():
            cnt_ref[d] = cntb

        @pl.when(b == 1)
        def _():
            cnt_ref[d] = cnt_ref[d] + cntb

        col = pl.ds(fb * _FB, _FB)
        for g in range(_NG):
            contrib = jnp.zeros((_NBG, _FB), jnp.float32)
            for h in range(_NH):
                feat = feat_refs[(d * _NG + g) * _NH + h][0]
                feat = feat.reshape(_FB, _HH)
                oh = onehot[:, h * _HH:(h + 1) * _HH]
                contrib += lax.dot_general(
                    oh, feat, dims, preferred_element_type=jnp.float32)

            @pl.when(b == 0)
            def _():
                out_refs[g][d, :, col] = contrib

            @pl.when(b == 1)
            def _():
                acc = out_refs[g][d, :, col] + contrib
                cnt = cnt_ref[d]
                out_refs[g][d, :, col] = jnp.where(cnt > 0.0,
                                                   acc / jnp.maximum(cnt, 1.0),
                                                   jnp.inf)


def _tc_reduce(labels, fs, ft):
    nfb = _GW // _FB
    feat_specs = []
    for d in range(2):
        for g in range(_NG):
            for h in range(_NH):
                feat_specs.append(
                    pl.BlockSpec((1, _FB, _HH),
                                 lambda fb, b, g=g, h=h:
                                 (b, g * nfb + fb, h)))
    outs = pl.pallas_call(
        _tc_body,
        grid=(nfb, 2),
        in_specs=[pl.BlockSpec((4, 1, _HW), lambda fb, b: (0, 0, 0))]
        + feat_specs,
        out_specs=[pl.BlockSpec((2, _NBG, _GW), lambda fb, b: (0, 0, 0))
                   for _ in range(_NG)],
        out_shape=[jax.ShapeDtypeStruct((2, _NBG, _GW), jnp.float32)
                   for _ in range(_NG)],
        scratch_shapes=[pltpu.VMEM((2, _NBG, 1), jnp.float32)],
        compiler_params=pltpu.CompilerParams(
            dimension_semantics=("parallel", "arbitrary")),
    )(labels, *([fs] * _NG * _NH), *([ft] * _NG * _NH))
    return jnp.concatenate(outs, axis=2)


def kernel(feat_source, gt_source, feat_target, gt_target):
    fs = feat_source.reshape(2, _F, _HW)
    ft = feat_target.reshape(2, _F, _HW)
    gs_rows = gt_source.astype(jnp.int32).reshape(2 * _ROWS_PER_BATCH, 128)
    gt_rows = gt_target.astype(jnp.int32).reshape(2 * _ROWS_PER_BATCH, 128)
    labels = _sc_labels(gs_rows, gt_rows).reshape(4, 1, _HW)
    return _tc_reduce(labels, fs, ft)


# speedup vs baseline: 1.0109x; 1.0014x over previous
import functools

import jax
import jax.numpy as jnp
from jax import lax
from jax.experimental import pallas as pl
from jax.experimental.pallas import tpu as pltpu
from jax.experimental.pallas import tpu_sc as plsc

_BG = (0, 1, 2, 3, 4, 8, 9, 10)
_NBG = 8
_F = 2048
_HW = 8192
_NPOS = 16384
_FB = 128
_NG = 2
_NH = 8
_GW = _F // _NG
_HH = _HW // _NH

_ROWS_PER_BATCH = 4096
_CHUNK = 512
_NW = 32
_L = 16


def _sc_labels_body(gs_hbm, gt_hbm, out_hbm, idx_v, rows_v, lab_v):
    wid = lax.axis_index("s") * 2 + lax.axis_index("c")
    p0 = wid * _CHUNK
    iota = lax.broadcasted_iota(jnp.int32, (_L,), 0)
    for d in range(2):
        src = gs_hbm if d == 0 else gt_hbm
        for n in range(2):
            q = p0 + 16 * (n * _L + iota)
            b = q >> 13
            i = (q >> 7) & 63
            jc = (q >> 4) & 7
            idx_v[pl.ds(n * _L, _L)] = (b << 12) + (i << 6) + jc
        pltpu.sync_copy(src.at[idx_v], rows_v)
        for m in range(_CHUNK // _L):
            row = jnp.full((_L,), m, jnp.int32)
            lab_v[pl.ds(m * _L, _L)] = plsc.load_gather(rows_v, [row, 8 * iota])
        pltpu.sync_copy(lab_v, out_hbm.at[pl.ds(d * _NPOS + p0, _CHUNK)])


def _sc_labels(gs_rows, gt_rows):
    mesh = plsc.VectorSubcoreMesh(core_axis_name="c", subcore_axis_name="s")
    kern = pl.kernel(
        _sc_labels_body,
        mesh=mesh,
        compiler_params=pltpu.CompilerParams(needs_layout_passes=False),
        out_type=jax.ShapeDtypeStruct((2 * _NPOS,), jnp.int32),
        scratch_types=[
            pltpu.VMEM((32,), jnp.int32),
            pltpu.VMEM((32, 128), jnp.int32),
            pltpu.VMEM((_CHUNK,), jnp.int32),
        ],
    )
    return kern(gs_rows, gt_rows)


def _tc_body(lab_ref, *refs):
    feat_refs = refs[:2 * _NG * _NH]
    out_refs = refs[2 * _NG * _NH:2 * _NG * _NH + _NG]
    cnt_ref = refs[-1]
    b = pl.program_id(1)
    fb = pl.program_id(0)
    cls = lax.broadcasted_iota(jnp.int32, (_NBG, 1), 0)
    bg = jnp.where(cls < 5, cls, cls + 3)
    dims = (((1,), (1,)), ((), ()))
    for d in range(2):
        labs = lab_ref[pl.ds(2 * d + b, 1), 0, :]
        onehot = (labs == bg).astype(jnp.float32)
        cntb = jnp.sum(onehot, axis=1, keepdims=True)

        @pl.when(b == 0)
        def _():
            cnt_ref[d] = cntb

        @pl.when(b == 1)
        def _():
            cnt_ref[d] = cnt_ref[d] + cntb

        col = pl.ds(fb * _FB, _FB)
        for g in range(_NG):
            contrib = jnp.zeros((_NBG, _FB), jnp.float32)
            for h in range(_NH):
                feat = feat_refs[(d * _NG + g) * _NH + h][0]
                feat = feat.reshape(_FB, _HH)
                oh = onehot[:, h * _HH:(h + 1) * _HH]
                contrib += lax.dot_general(
                    oh, feat, dims, preferred_element_type=jnp.float32)

            @pl.when(b == 0)
            def _():
                out_refs[g][d, :, col] = contrib

            @pl.when(b == 1)
            def _():
                acc = out_refs[g][d, :, col] + contrib
                cnt = cnt_ref[d]
                out_refs[g][d, :, col] = jnp.where(cnt > 0.0,
                                                   acc / jnp.maximum(cnt, 1.0),
                                                   jnp.inf)


def _tc_reduce(labels, fs, ft):
    nfb = _GW // _FB
    feat_specs = []
    for d in range(2):
        for g in range(_NG):
            for h in range(_NH):
                feat_specs.append(
                    pl.BlockSpec((1, _FB, _HH),
                                 lambda fb, b, g=g, h=h:
                                 (b, g * nfb + fb, h)))
    outs = pl.pallas_call(
        _tc_body,
        grid=(nfb, 2),
        in_specs=[pl.BlockSpec((4, 1, _HW), lambda fb, b: (0, 0, 0))]
        + feat_specs,
        out_specs=[pl.BlockSpec((2, _NBG, _GW), lambda fb, b: (0, 0, 0))
                   for _ in range(_NG)],
        out_shape=[jax.ShapeDtypeStruct((2, _NBG, _GW), jnp.float32)
                   for _ in range(_NG)],
        scratch_shapes=[pltpu.VMEM((2, _NBG, 1), jnp.float32)],
        compiler_params=pltpu.CompilerParams(
            dimension_semantics=("parallel", "arbitrary")),
    )(labels, *([fs] * _NG * _NH), *([ft] * _NG * _NH))
    return jnp.concatenate(outs, axis=2)


def kernel(feat_source, gt_source, feat_target, gt_target):
    fs = feat_source.reshape(2, _F, _HW)
    ft = feat_target.reshape(2, _F, _HW)
    gs_rows = gt_source.astype(jnp.int32).reshape(2 * _ROWS_PER_BATCH, 128)
    gt_rows = gt_target.astype(jnp.int32).reshape(2 * _ROWS_PER_BATCH, 128)
    labels = _sc_labels(gs_rows, gt_rows).reshape(4, 1, _HW)
    return _tc_reduce(labels, fs, ft)
